# baseline (speedup 1.0000x reference)
import functools

import jax
import jax.numpy as jnp
from jax.experimental import pallas as pl

_EPS = 1e-8


def _split3_last(x):
    hi = x.astype(jnp.bfloat16)
    r1 = x - hi.astype(jnp.float32)
    mid = r1.astype(jnp.bfloat16)
    lo = (r1 - mid.astype(jnp.float32)).astype(jnp.bfloat16)
    return hi, mid, lo


def _chamfer_kernel(lhs_ref, rhs_ref, pT_ref, g_ref, gTb_ref, p3b_ref,
                    out_ref, *, tm: int, m: int, n: int):
    rhs = rhs_ref[0]
    gTb = gTb_ref[0]

    num_tiles = m // tm
    dims = (((1,), (0,)), ((), ()))

    def body(i, carry):
        fsum, colmin_a, colselp = carry
        lhs = lhs_ref[0, pl.ds(i * tm, tm), :]
        pf = pT_ref[0, pl.ds(i * tm, tm), :]
        d2a = jax.lax.dot_general(
            lhs, rhs, dimension_numbers=dims,
            preferred_element_type=jnp.float32)

        rowmin_a = jnp.min(d2a, axis=1, keepdims=True)
        rowhot = (d2a == rowmin_a).astype(jnp.bfloat16)
        selg = jax.lax.dot_general(
            rowhot, gTb, dimension_numbers=dims,
            preferred_element_type=jnp.float32)
        dg = selg - pf
        d2row = jnp.sum(dg * dg, axis=1)
        fsum = fsum + jnp.sum(jnp.sqrt(d2row + _EPS))

        tile_cmin = jnp.min(d2a, axis=0, keepdims=True)
        colhot = (d2a == tile_cmin).astype(jnp.bfloat16)
        p3b = p3b_ref[0, :, pl.ds(i * tm, tm)]
        tile_selp = jax.lax.dot_general(
            p3b, colhot, dimension_numbers=dims,
            preferred_element_type=jnp.float32)
        take_new = tile_cmin < colmin_a
        colselp = jnp.where(take_new, tile_selp, colselp)
        colmin_a = jnp.where(take_new, tile_cmin, colmin_a)
        return fsum, colmin_a, colselp

    init = (jnp.float32(0.0),
            jnp.full((1, n), jnp.inf, dtype=jnp.float32),
            jnp.zeros((3, n), dtype=jnp.float32))
    fsum, _, colselp = jax.lax.fori_loop(0, num_tiles, body, init)
    dpx = colselp[0:1, :] - g_ref[0, 0:1, :]
    dpy = colselp[1:2, :] - g_ref[0, 1:2, :]
    dpz = colselp[2:3, :] - g_ref[0, 2:3, :]
    d2col = dpx * dpx + dpy * dpy + dpz * dpz
    bsum = jnp.sum(jnp.sqrt(d2col + _EPS))
    row = jax.lax.broadcasted_iota(jnp.int32, (8, 128), 0)
    col = jax.lax.broadcasted_iota(jnp.int32, (8, 128), 1)
    out = jnp.where((row == 0) & (col == 0), fsum,
                    jnp.where((row == 0) & (col == 1), bsum, 0.0))
    out_ref[0] = out


@jax.jit
def kernel(predict_pc, gt_pc):
    b, _, m = predict_pc.shape
    n = gt_pc.shape[2]
    tm = 2048
    p3 = predict_pc[:, :3, :]
    g = gt_pc[:, :3, :]
    pT = jnp.transpose(p3, (0, 2, 1))
    gT = jnp.transpose(g, (0, 2, 1))

    p_sq = jnp.sum(pT * pT, axis=2)
    g_sq = jnp.sum(g * g, axis=1)
    ph, pm_, plo = _split3_last(p_sq)
    gh, gm, glo = _split3_last(g_sq)
    onesM = jnp.ones((b, m), jnp.bfloat16)
    onesN = jnp.ones((b, n), jnp.bfloat16)
    zeros7M = jnp.zeros((b, m, 7), jnp.bfloat16)
    zeros7N = jnp.zeros((b, 7, n), jnp.bfloat16)
    lhs = jnp.concatenate(
        [(-2.0 * pT).astype(jnp.bfloat16)] +
        [x[:, :, None] for x in (ph, pm_, plo, onesM, onesM, onesM)] +
        [zeros7M], axis=2)
    rhs = jnp.concatenate(
        [g.astype(jnp.bfloat16)] +
        [x[:, None, :] for x in (onesN, onesN, onesN, gh, gm, glo)] +
        [zeros7N], axis=1)

    out = pl.pallas_call(
        functools.partial(_chamfer_kernel, tm=tm, m=m, n=n),
        grid=(b,),
        in_specs=[
            pl.BlockSpec((1, m, 16), lambda i: (i, 0, 0)),
            pl.BlockSpec((1, 16, n), lambda i: (i, 0, 0)),
            pl.BlockSpec((1, m, 3), lambda i: (i, 0, 0)),
            pl.BlockSpec((1, 3, n), lambda i: (i, 0, 0)),
            pl.BlockSpec((1, n, 3), lambda i: (i, 0, 0)),
            pl.BlockSpec((1, 3, m), lambda i: (i, 0, 0)),
        ],
        out_specs=pl.BlockSpec((1, 8, 128), lambda i: (i, 0, 0)),
        out_shape=jax.ShapeDtypeStruct((b, 8, 128), jnp.float32),
    )(lhs, rhs, pT, g, gT.astype(jnp.bfloat16), p3.astype(jnp.bfloat16))
    forward = jnp.sum(out[:, 0, 0]) / (b * m)
    backward = jnp.sum(out[:, 0, 1]) / (b * n)
    return forward + backward


# speedup vs baseline: 1.5580x; 1.5580x over previous
import functools

import jax
import jax.numpy as jnp
from jax.experimental import pallas as pl

_EPS = 1e-8


def _chamfer_kernel(pT_ref, g_ref, gT_ref, p3_ref, out_ref, *,
                    tm: int, m: int, n: int):
    gx = g_ref[0, 0:1, :]
    gy = g_ref[0, 1:2, :]
    gz = g_ref[0, 2:3, :]
    g_sq = gx * gx + gy * gy + gz * gz
    gb = g_ref[0].astype(jnp.bfloat16)
    gTb = gT_ref[0].astype(jnp.bfloat16)

    num_tiles = m // tm
    dims = (((1,), (0,)), ((), ()))

    def body(i, carry):
        fsum, colmin_a, colselp = carry
        pf = pT_ref[0, pl.ds(i * tm, tm), :]
        p3 = p3_ref[0, :, pl.ds(i * tm, tm)]
        px = pf[:, 0:1]
        py = pf[:, 1:2]
        pz = pf[:, 2:3]
        p_sq = px * px + py * py + pz * pz
        pb = pf.astype(jnp.bfloat16)
        cross = jax.lax.dot_general(
            pb, gb, dimension_numbers=dims,
            preferred_element_type=jnp.float32)
        d2a = p_sq - 2.0 * cross + g_sq

        rowmin_a = jnp.min(d2a, axis=1, keepdims=True)
        rowhot = (d2a == rowmin_a).astype(jnp.bfloat16)
        selgT = jax.lax.dot_general(
            gb, rowhot, dimension_numbers=(((1,), (1,)), ((), ())),
            preferred_element_type=jnp.float32)
        dgx = selgT[0:1, :] - p3[0:1, :]
        dgy = selgT[1:2, :] - p3[1:2, :]
        dgz = selgT[2:3, :] - p3[2:3, :]
        d2row = dgx * dgx + dgy * dgy + dgz * dgz
        fsum = fsum + jnp.sum(jnp.sqrt(d2row + _EPS))

        tile_cmin = jnp.min(d2a, axis=0, keepdims=True)
        colhot = (d2a == tile_cmin).astype(jnp.bfloat16)
        tile_selp = jax.lax.dot_general(
            p3.astype(jnp.bfloat16), colhot, dimension_numbers=dims,
            preferred_element_type=jnp.float32)
        take_new = tile_cmin < colmin_a
        colselp = jnp.where(take_new, tile_selp, colselp)
        colmin_a = jnp.where(take_new, tile_cmin, colmin_a)
        return fsum, colmin_a, colselp

    init = (jnp.float32(0.0),
            jnp.full((1, n), jnp.inf, dtype=jnp.float32),
            jnp.zeros((3, n), dtype=jnp.float32))
    fsum, _, colselp = jax.lax.fori_loop(0, num_tiles, body, init)
    dpx = colselp[0:1, :] - gx
    dpy = colselp[1:2, :] - gy
    dpz = colselp[2:3, :] - gz
    d2col = dpx * dpx + dpy * dpy + dpz * dpz
    bsum = jnp.sum(jnp.sqrt(d2col + _EPS))
    row = jax.lax.broadcasted_iota(jnp.int32, (8, 128), 0)
    col = jax.lax.broadcasted_iota(jnp.int32, (8, 128), 1)
    out = jnp.where((row == 0) & (col == 0), fsum,
                    jnp.where((row == 0) & (col == 1), bsum, 0.0))
    out_ref[0] = out


@jax.jit
def kernel(predict_pc, gt_pc):
    b, _, m = predict_pc.shape
    n = gt_pc.shape[2]
    tm = 2048
    p3 = predict_pc[:, :3, :]
    g = gt_pc[:, :3, :]
    pT = jnp.transpose(p3, (0, 2, 1))
    gT = jnp.transpose(g, (0, 2, 1))
    out = pl.pallas_call(
        functools.partial(_chamfer_kernel, tm=tm, m=m, n=n),
        grid=(b,),
        in_specs=[
            pl.BlockSpec((1, m, 3), lambda i: (i, 0, 0)),
            pl.BlockSpec((1, 3, n), lambda i: (i, 0, 0)),
            pl.BlockSpec((1, n, 3), lambda i: (i, 0, 0)),
            pl.BlockSpec((1, 3, m), lambda i: (i, 0, 0)),
        ],
        out_specs=pl.BlockSpec((1, 8, 128), lambda i: (i, 0, 0)),
        out_shape=jax.ShapeDtypeStruct((b, 8, 128), jnp.float32),
    )(pT, g, gT, p3)
    forward = jnp.sum(out[:, 0, 0]) / (b * m)
    backward = jnp.sum(out[:, 0, 1]) / (b * n)
    return forward + backward


# speedup vs baseline: 1.6051x; 1.0302x over previous
import functools

import jax
import jax.numpy as jnp
from jax.experimental import pallas as pl

_EPS = 1e-8


def _chamfer_kernel(pT_ref, g_ref, p3_ref, out_ref, *,
                    tm: int, m: int, n: int):
    gx = g_ref[0, 0:1, :]
    gy = g_ref[0, 1:2, :]
    gz = g_ref[0, 2:3, :]
    g_sq = gx * gx + gy * gy + gz * gz
    gb = g_ref[0].astype(jnp.bfloat16)

    num_tiles = m // tm
    dims = (((1,), (0,)), ((), ()))

    def body(i, carry):
        fsum, colmin_a, colselp = carry
        pf = pT_ref[0, pl.ds(i * tm, tm), :]
        p3 = p3_ref[0, :, pl.ds(i * tm, tm)]
        px = pf[:, 0:1]
        py = pf[:, 1:2]
        pz = pf[:, 2:3]
        p_sq = px * px + py * py + pz * pz
        pb = pf.astype(jnp.bfloat16)
        cross = jax.lax.dot_general(
            pb, gb, dimension_numbers=dims,
            preferred_element_type=jnp.float32)
        d2a = p_sq - 2.0 * cross + g_sq

        rowmin_a = jnp.min(d2a, axis=1, keepdims=True)
        rowhot = (d2a == rowmin_a).astype(jnp.bfloat16)
        selgT = jax.lax.dot_general(
            gb, rowhot, dimension_numbers=(((1,), (1,)), ((), ())),
            preferred_element_type=jnp.float32)
        dgx = selgT[0:1, :] - p3[0:1, :]
        dgy = selgT[1:2, :] - p3[1:2, :]
        dgz = selgT[2:3, :] - p3[2:3, :]
        d2row = dgx * dgx + dgy * dgy + dgz * dgz
        fsum = fsum + jnp.sum(jnp.sqrt(d2row + _EPS))

        tile_cmin = jnp.min(d2a, axis=0, keepdims=True)
        colhot = (d2a == tile_cmin).astype(jnp.bfloat16)
        tile_selp = jax.lax.dot_general(
            p3.astype(jnp.bfloat16), colhot, dimension_numbers=dims,
            preferred_element_type=jnp.float32)
        take_new = tile_cmin < colmin_a
        colselp = jnp.where(take_new, tile_selp, colselp)
        colmin_a = jnp.where(take_new, tile_cmin, colmin_a)
        return fsum, colmin_a, colselp

    init = (jnp.float32(0.0),
            jnp.full((1, n), jnp.inf, dtype=jnp.float32),
            jnp.zeros((3, n), dtype=jnp.float32))
    carry = init
    for i in range(num_tiles):
        carry = body(i, carry)
    fsum, _, colselp = carry
    dpx = colselp[0:1, :] - gx
    dpy = colselp[1:2, :] - gy
    dpz = colselp[2:3, :] - gz
    d2col = dpx * dpx + dpy * dpy + dpz * dpz
    bsum = jnp.sum(jnp.sqrt(d2col + _EPS))
    row = jax.lax.broadcasted_iota(jnp.int32, (8, 128), 0)
    col = jax.lax.broadcasted_iota(jnp.int32, (8, 128), 1)
    out = jnp.where((row == 0) & (col == 0), fsum,
                    jnp.where((row == 0) & (col == 1), bsum, 0.0))
    out_ref[0] = out


@jax.jit
def kernel(predict_pc, gt_pc):
    b, _, m = predict_pc.shape
    n = gt_pc.shape[2]
    tm = 2048
    p3 = predict_pc[:, :3, :]
    g = gt_pc[:, :3, :]
    pT = jnp.transpose(p3, (0, 2, 1))
    out = pl.pallas_call(
        functools.partial(_chamfer_kernel, tm=tm, m=m, n=n),
        grid=(b,),
        in_specs=[
            pl.BlockSpec((1, m, 3), lambda i: (i, 0, 0)),
            pl.BlockSpec((1, 3, n), lambda i: (i, 0, 0)),
            pl.BlockSpec((1, 3, m), lambda i: (i, 0, 0)),
        ],
        out_specs=pl.BlockSpec((1, 8, 128), lambda i: (i, 0, 0)),
        out_shape=jax.ShapeDtypeStruct((b, 8, 128), jnp.float32),
    )(pT, g, p3)
    forward = jnp.sum(out[:, 0, 0]) / (b * m)
    backward = jnp.sum(out[:, 0, 1]) / (b * n)
    return forward + backward


# speedup vs baseline: 1.6653x; 1.0375x over previous
import functools

import jax
import jax.numpy as jnp
from jax.experimental import pallas as pl

_EPS = 1e-8


def _chamfer_kernel(pT_ref, g_ref, p3_ref, out_ref, *,
                    tm: int, m: int, n: int):
    gx = g_ref[0, 0:1, :]
    gy = g_ref[0, 1:2, :]
    gz = g_ref[0, 2:3, :]
    g_sq = gx * gx + gy * gy + gz * gz
    gb = g_ref[0].astype(jnp.bfloat16)

    num_tiles = m // tm
    dims = (((1,), (0,)), ((), ()))

    def body(i, carry):
        fsum, colmin_a, colselp = carry
        pf = pT_ref[0, pl.ds(i * tm, tm), :]
        p3 = p3_ref[0, :, pl.ds(i * tm, tm)]
        px = pf[:, 0:1]
        py = pf[:, 1:2]
        pz = pf[:, 2:3]
        p_sq = px * px + py * py + pz * pz
        pb = pf.astype(jnp.bfloat16)
        cross = jax.lax.dot_general(
            pb, gb, dimension_numbers=dims,
            preferred_element_type=jnp.float32)
        d2a = p_sq - 2.0 * cross + g_sq

        rowmin_a = jnp.min(d2a, axis=1, keepdims=True)
        rowhot = (d2a == rowmin_a).astype(jnp.bfloat16)
        selgT = jax.lax.dot_general(
            gb, rowhot, dimension_numbers=(((1,), (1,)), ((), ())),
            preferred_element_type=jnp.float32)
        dgx = selgT[0:1, :] - p3[0:1, :]
        dgy = selgT[1:2, :] - p3[1:2, :]
        dgz = selgT[2:3, :] - p3[2:3, :]
        d2row = dgx * dgx + dgy * dgy + dgz * dgz
        fsum = fsum + jnp.sum(jnp.sqrt(d2row + _EPS))

        tile_cmin = jnp.min(d2a, axis=0, keepdims=True)
        colhot = (d2a == tile_cmin).astype(jnp.bfloat16)
        tile_selp = jax.lax.dot_general(
            p3.astype(jnp.bfloat16), colhot, dimension_numbers=dims,
            preferred_element_type=jnp.float32)
        take_new = tile_cmin < colmin_a
        colselp = jnp.where(take_new, tile_selp, colselp)
        colmin_a = jnp.where(take_new, tile_cmin, colmin_a)
        return fsum, colmin_a, colselp

    init = (jnp.float32(0.0),
            jnp.full((1, n), jnp.inf, dtype=jnp.float32),
            jnp.zeros((3, n), dtype=jnp.float32))
    carry = init
    for i in range(num_tiles):
        carry = body(i, carry)
    fsum, _, colselp = carry
    dpx = colselp[0:1, :] - gx
    dpy = colselp[1:2, :] - gy
    dpz = colselp[2:3, :] - gz
    d2col = dpx * dpx + dpy * dpy + dpz * dpz
    bsum = jnp.sum(jnp.sqrt(d2col + _EPS))
    row = jax.lax.broadcasted_iota(jnp.int32, (8, 128), 0)
    col = jax.lax.broadcasted_iota(jnp.int32, (8, 128), 1)
    out = jnp.where((row == 0) & (col == 0), fsum,
                    jnp.where((row == 0) & (col == 1), bsum, 0.0))
    out_ref[0] = out


@jax.jit
def kernel(predict_pc, gt_pc):
    b, _, m = predict_pc.shape
    n = gt_pc.shape[2]
    tm = 1024
    p3 = predict_pc[:, :3, :]
    g = gt_pc[:, :3, :]
    pT = jnp.transpose(p3, (0, 2, 1))
    out = pl.pallas_call(
        functools.partial(_chamfer_kernel, tm=tm, m=m, n=n),
        grid=(b,),
        in_specs=[
            pl.BlockSpec((1, m, 3), lambda i: (i, 0, 0)),
            pl.BlockSpec((1, 3, n), lambda i: (i, 0, 0)),
            pl.BlockSpec((1, 3, m), lambda i: (i, 0, 0)),
        ],
        out_specs=pl.BlockSpec((1, 8, 128), lambda i: (i, 0, 0)),
        out_shape=jax.ShapeDtypeStruct((b, 8, 128), jnp.float32),
    )(pT, g, p3)
    forward = jnp.sum(out[:, 0, 0]) / (b * m)
    backward = jnp.sum(out[:, 0, 1]) / (b * n)
    return forward + backward


# speedup vs baseline: 1.7027x; 1.0225x over previous
import functools

import jax
import jax.numpy as jnp
from jax.experimental import pallas as pl

_EPS = 1e-8


def _chamfer_kernel(pT_ref, g_ref, p3_ref, out_ref, *,
                    tm: int, m: int, n: int):
    gx = g_ref[0, 0:1, :]
    gy = g_ref[0, 1:2, :]
    gz = g_ref[0, 2:3, :]
    g_sq = gx * gx + gy * gy + gz * gz
    gb = g_ref[0].astype(jnp.bfloat16)

    num_tiles = m // tm
    dims = (((1,), (0,)), ((), ()))

    def body(i, carry):
        fsum, colmin_a, colselp = carry
        pf = pT_ref[0, pl.ds(i * tm, tm), :]
        p3 = p3_ref[0, :, pl.ds(i * tm, tm)]
        px = pf[:, 0:1]
        py = pf[:, 1:2]
        pz = pf[:, 2:3]
        p_sq = px * px + py * py + pz * pz
        pb = pf.astype(jnp.bfloat16)
        cross = jax.lax.dot_general(
            pb, gb, dimension_numbers=dims,
            preferred_element_type=jnp.float32)
        d2a = p_sq - 2.0 * cross + g_sq

        rowmin_a = jnp.min(d2a, axis=1, keepdims=True)
        rowhot = (d2a == rowmin_a).astype(jnp.bfloat16)
        selgT = jax.lax.dot_general(
            gb, rowhot, dimension_numbers=(((1,), (1,)), ((), ())),
            preferred_element_type=jnp.float32)
        dgx = selgT[0:1, :] - p3[0:1, :]
        dgy = selgT[1:2, :] - p3[1:2, :]
        dgz = selgT[2:3, :] - p3[2:3, :]
        d2row = dgx * dgx + dgy * dgy + dgz * dgz
        fsum = fsum + jnp.sum(jnp.sqrt(d2row + _EPS))

        tile_cmin = jnp.min(d2a, axis=0, keepdims=True)
        colhot = (d2a == tile_cmin).astype(jnp.bfloat16)
        tile_selp = jax.lax.dot_general(
            p3.astype(jnp.bfloat16), colhot, dimension_numbers=dims,
            preferred_element_type=jnp.float32)
        take_new = tile_cmin < colmin_a
        colselp = jnp.where(take_new, tile_selp, colselp)
        colmin_a = jnp.where(take_new, tile_cmin, colmin_a)
        return fsum, colmin_a, colselp

    init = (jnp.float32(0.0),
            jnp.full((1, n), jnp.inf, dtype=jnp.float32),
            jnp.zeros((3, n), dtype=jnp.float32))
    carry = init
    for i in range(num_tiles):
        carry = body(i, carry)
    fsum, _, colselp = carry
    dpx = colselp[0:1, :] - gx
    dpy = colselp[1:2, :] - gy
    dpz = colselp[2:3, :] - gz
    d2col = dpx * dpx + dpy * dpy + dpz * dpz
    bsum = jnp.sum(jnp.sqrt(d2col + _EPS))
    row = jax.lax.broadcasted_iota(jnp.int32, (8, 128), 0)
    col = jax.lax.broadcasted_iota(jnp.int32, (8, 128), 1)
    out = jnp.where((row == 0) & (col == 0), fsum,
                    jnp.where((row == 0) & (col == 1), bsum, 0.0))
    out_ref[0] = out


@jax.jit
def kernel(predict_pc, gt_pc):
    b, _, m = predict_pc.shape
    n = gt_pc.shape[2]
    tm = 512
    p3 = predict_pc[:, :3, :]
    g = gt_pc[:, :3, :]
    pT = jnp.transpose(p3, (0, 2, 1))
    out = pl.pallas_call(
        functools.partial(_chamfer_kernel, tm=tm, m=m, n=n),
        grid=(b,),
        in_specs=[
            pl.BlockSpec((1, m, 3), lambda i: (i, 0, 0)),
            pl.BlockSpec((1, 3, n), lambda i: (i, 0, 0)),
            pl.BlockSpec((1, 3, m), lambda i: (i, 0, 0)),
        ],
        out_specs=pl.BlockSpec((1, 8, 128), lambda i: (i, 0, 0)),
        out_shape=jax.ShapeDtypeStruct((b, 8, 128), jnp.float32),
    )(pT, g, p3)
    forward = jnp.sum(out[:, 0, 0]) / (b * m)
    backward = jnp.sum(out[:, 0, 1]) / (b * n)
    return forward + backward


# speedup vs baseline: 1.7462x; 1.0256x over previous
import functools

import jax
import jax.numpy as jnp
from jax.experimental import pallas as pl

_EPS = 1e-8


def _chamfer_kernel(pT_ref, g_ref, p3_ref, out_ref, *,
                    tm: int, m: int, n: int):
    gx = g_ref[0, 0:1, :]
    gy = g_ref[0, 1:2, :]
    gz = g_ref[0, 2:3, :]
    g_sq = gx * gx + gy * gy + gz * gz
    gb = g_ref[0].astype(jnp.bfloat16)

    num_tiles = m // tm
    dims = (((1,), (0,)), ((), ()))

    def body(i, carry):
        fsum, colmin_a, colselp = carry
        pf = pT_ref[0, pl.ds(i * tm, tm), :]
        p3 = p3_ref[0, :, pl.ds(i * tm, tm)]
        px = pf[:, 0:1]
        py = pf[:, 1:2]
        pz = pf[:, 2:3]
        p_sq = px * px + py * py + pz * pz
        pb = pf.astype(jnp.bfloat16)
        cross = jax.lax.dot_general(
            pb, gb, dimension_numbers=dims,
            preferred_element_type=jnp.float32)
        d2a = p_sq - 2.0 * cross + g_sq

        rowmin_a = jnp.min(d2a, axis=1, keepdims=True)
        rowhot = (d2a == rowmin_a).astype(jnp.bfloat16)
        selgT = jax.lax.dot_general(
            gb, rowhot, dimension_numbers=(((1,), (1,)), ((), ())),
            preferred_element_type=jnp.float32)
        dgx = selgT[0:1, :] - p3[0:1, :]
        dgy = selgT[1:2, :] - p3[1:2, :]
        dgz = selgT[2:3, :] - p3[2:3, :]
        d2row = dgx * dgx + dgy * dgy + dgz * dgz
        fsum = fsum + jnp.sum(jnp.sqrt(d2row + _EPS))

        tile_cmin = jnp.min(d2a, axis=0, keepdims=True)
        colhot = (d2a == tile_cmin).astype(jnp.bfloat16)
        tile_selp = jax.lax.dot_general(
            p3.astype(jnp.bfloat16), colhot, dimension_numbers=dims,
            preferred_element_type=jnp.float32)
        take_new = tile_cmin < colmin_a
        colselp = jnp.where(take_new, tile_selp, colselp)
        colmin_a = jnp.where(take_new, tile_cmin, colmin_a)
        return fsum, colmin_a, colselp

    init = (jnp.float32(0.0),
            jnp.full((1, n), jnp.inf, dtype=jnp.float32),
            jnp.zeros((3, n), dtype=jnp.float32))
    carry = init
    for i in range(num_tiles):
        carry = body(i, carry)
    fsum, _, colselp = carry
    dpx = colselp[0:1, :] - gx
    dpy = colselp[1:2, :] - gy
    dpz = colselp[2:3, :] - gz
    d2col = dpx * dpx + dpy * dpy + dpz * dpz
    bsum = jnp.sum(jnp.sqrt(d2col + _EPS))
    row = jax.lax.broadcasted_iota(jnp.int32, (8, 128), 0)
    col = jax.lax.broadcasted_iota(jnp.int32, (8, 128), 1)
    out = jnp.where((row == 0) & (col == 0), fsum,
                    jnp.where((row == 0) & (col == 1), bsum, 0.0))
    out_ref[0] = out


@jax.jit
def kernel(predict_pc, gt_pc):
    b, _, m = predict_pc.shape
    n = gt_pc.shape[2]
    tm = 256
    p3 = predict_pc[:, :3, :]
    g = gt_pc[:, :3, :]
    pT = jnp.transpose(p3, (0, 2, 1))
    out = pl.pallas_call(
        functools.partial(_chamfer_kernel, tm=tm, m=m, n=n),
        grid=(b,),
        in_specs=[
            pl.BlockSpec((1, m, 3), lambda i: (i, 0, 0)),
            pl.BlockSpec((1, 3, n), lambda i: (i, 0, 0)),
            pl.BlockSpec((1, 3, m), lambda i: (i, 0, 0)),
        ],
        out_specs=pl.BlockSpec((1, 8, 128), lambda i: (i, 0, 0)),
        out_shape=jax.ShapeDtypeStruct((b, 8, 128), jnp.float32),
    )(pT, g, p3)
    forward = jnp.sum(out[:, 0, 0]) / (b * m)
    backward = jnp.sum(out[:, 0, 1]) / (b * n)
    return forward + backward
